# baseline (speedup 1.0000x reference)
import jax
import jax.numpy as jnp
from jax import lax
from jax.experimental import pallas as pl
from jax.experimental.pallas import tpu as pltpu
from jax.experimental.pallas import tpu_sc as plsc

BS, M, D, DE = 16, 65536, 1024, 64
K = 32
BM = 8192
NB = M // BM
NEG_INF = float('-inf')
GROUPS = M // 256
NL1 = M // 16


def _scores_kernel(x_ref, y_ref, w_ref, b_ref, s_ref, k_ref, out_ref, q_scr):
    @pl.when((pl.program_id(0) == 0) & (pl.program_id(1) == 0))
    def _():
        xw = lax.dot_general(x_ref[...], w_ref[:, :D], (((1,), (1,)), ((), ())),
                             preferred_element_type=jnp.float32,
                         precision=lax.Precision.HIGHEST)
        yw = lax.dot_general(y_ref[...], w_ref[:, D:], (((1,), (1,)), ((), ())),
                             preferred_element_type=jnp.float32,
                         precision=lax.Precision.HIGHEST)
        q = xw + yw + b_ref[...]
        n = jnp.sqrt(jnp.sum(q * q, axis=-1, keepdims=True))
        q_scr[...] = q / jnp.maximum(n, 1e-6)

    b = pl.program_id(0)
    qb = q_scr[pl.ds(b, 1), :].astype(jnp.bfloat16)
    kb = k_ref[0].astype(jnp.bfloat16)
    s = lax.dot_general(qb, kb, (((1,), (1,)), ((), ())),
                        preferred_element_type=jnp.float32)[0]
    out_ref[...] = jnp.where(s_ref[0, 0, 0] > 0, s, NEG_INF)[None, None, None, :]


def _scores_call(x, y_wm, W_q_em_w, W_q_em_b, em_S, em_K):
    return pl.pallas_call(
        _scores_kernel,
        grid=(BS, NB),
        in_specs=[
            pl.BlockSpec((BS, D), lambda b, m: (0, 0)),
            pl.BlockSpec((BS, D), lambda b, m: (0, 0)),
            pl.BlockSpec((DE, 2 * D), lambda b, m: (0, 0)),
            pl.BlockSpec((1, DE), lambda b, m: (0, 0)),
            pl.BlockSpec((1, 1, 1, BM), lambda b, m: (b, m, 0, 0)),
            pl.BlockSpec((1, BM, DE), lambda b, m: (b, m, 0)),
        ],
        out_specs=pl.BlockSpec((1, 1, 1, BM), lambda b, m: (b, m, 0, 0)),
        out_shape=jax.ShapeDtypeStruct((BS, NB, 1, BM), jnp.float32),
        scratch_shapes=[pltpu.VMEM((BS, DE), jnp.float32)],
    )(x, y_wm, W_q_em_w, W_q_em_b.reshape(1, DE),
      em_S.reshape(BS, NB, 1, BM), em_K).reshape(BS, M)


def _vgather(v, idx):
    return lax.gather(
        v, idx[:, None],
        lax.GatherDimensionNumbers(offset_dims=(), collapsed_slice_dims=(0,),
                                   start_index_map=(0,)),
        (1,), mode=lax.GatherScatterMode.PROMISE_IN_BOUNDS)


def _extract32(ref_v, ref_i, nv, iota):
    zeros = jnp.zeros((16,), jnp.int32)
    minus_inf = jnp.full((16,), NEG_INF, jnp.float32)

    def it(i, carry):
        rv0, rv1, ri0, ri1 = carry

        def scan(j, c):
            m, mj = c
            v = ref_v[pl.ds(j * 16, 16)]
            upd = v > m
            return jnp.where(upd, v, m), jnp.where(upd, j, mj)

        m, mj = lax.fori_loop(0, nv, scan, (minus_inf, zeros))
        sk, sv = plsc.sort_key_val(m, iota, descending=True)
        mx = _vgather(sk, zeros)
        lane = _vgather(sv, zeros)
        onehot = iota == lane
        jwin = _vgather(mj, lane)
        pos = jwin * 16 + lane
        gidx = pos if ref_i is None else plsc.load_gather(ref_i, [pos])
        plsc.store_scatter(ref_v, [pos], minus_inf, mask=onehot)
        rv0 = jnp.where(iota == i, mx, rv0)
        rv1 = jnp.where(iota == (i - 16), mx, rv1)
        ri0 = jnp.where(iota == i, gidx, ri0)
        ri1 = jnp.where(iota == (i - 16), gidx, ri1)
        return rv0, rv1, ri0, ri1

    return lax.fori_loop(0, K, it, (minus_inf, minus_inf, zeros, zeros))


def _sc_body(scores_hbm, emv_hbm, outs_hbm, outv_hbm,
             sc_v, lmax_v, l2_v, cand_v, cand_i, idx_v, val_v, rows2_v,
             out_v, sem):
    wid = lax.axis_index("s") * 2 + lax.axis_index("c")

    @pl.when(wid < BS)
    def _():
        b = wid
        pltpu.sync_copy(scores_hbm.at[b], sc_v)
        iota = lax.iota(jnp.int32, 16)

        def l1_body(g, carry):
            m = sc_v[pl.ds(g * 256, 16)]
            for i in range(1, 16):
                m = jnp.maximum(m, sc_v[pl.ds(g * 256 + i * 16, 16)])
            lmax_v[pl.ds(g * 16, 16)] = m
            return carry

        lax.fori_loop(0, GROUPS, l1_body, 0)

        def l2_body(s, carry):
            m = lmax_v[pl.ds(s * 256, 16)]
            for i in range(1, 16):
                m = jnp.maximum(m, lmax_v[pl.ds(s * 256 + i * 16, 16)])
            l2_v[pl.ds(s * 16, 16)] = m
            return carry

        lax.fori_loop(0, 16, l2_body, 0)

        _, _, w20, w21 = _extract32(l2_v, None, 16, iota)

        for w in range(K):
            src = w20 if w < 16 else w21
            jid = _vgather(src, jnp.full((16,), w % 16, jnp.int32))
            ids = (jid // 16) * 256 + (jid % 16) + iota * 16
            cand_v[pl.ds(w * 16, 16)] = plsc.load_gather(lmax_v, [ids])
            cand_i[pl.ds(w * 16, 16)] = ids

        _, _, w10, w11 = _extract32(cand_v, cand_i, K, iota)

        for w in range(K):
            src = w10 if w < 16 else w11
            jid = _vgather(src, jnp.full((16,), w % 16, jnp.int32))
            ids = (jid // 16) * 256 + (jid % 16) + iota * 16
            cand_v[pl.ds(w * 16, 16)] = plsc.load_gather(sc_v, [ids])
            cand_i[pl.ds(w * 16, 16)] = ids

        rv0, rv1, ri0, ri1 = _extract32(cand_v, cand_i, K, iota)

        val_v[pl.ds(0, 16)] = rv0
        val_v[pl.ds(16, 16)] = rv1
        idx_v[pl.ds(0, 16)] = (ri0 + b * M) >> 1
        idx_v[pl.ds(16, 16)] = (ri1 + b * M) >> 1

        pltpu.async_copy(emv_hbm.at[idx_v], rows2_v, sem).wait()
        for r in range(K):
            srci = ri0 if r < 16 else ri1
            idxr = _vgather(srci, jnp.full((16,), r % 16, jnp.int32))
            off = (idxr & 1) * DE
            rsplat = jnp.full((16,), r, jnp.int32)
            for k2 in range(DE // 16):
                out_v[pl.ds(r * DE + k2 * 16, 16)] = plsc.load_gather(
                    rows2_v, [rsplat, off + k2 * 16 + iota])
        pltpu.sync_copy(out_v, outv_hbm.at[b])
        pltpu.sync_copy(val_v, outs_hbm.at[b])


def _topk_gather_call(scores, em_V_flat):
    f = pl.kernel(
        _sc_body,
        out_type=(jax.ShapeDtypeStruct((BS, K), jnp.float32),
                  jax.ShapeDtypeStruct((BS, K * DE), jnp.float32)),
        mesh=plsc.VectorSubcoreMesh(core_axis_name="c", subcore_axis_name="s",
                                    num_cores=2, num_subcores=16),
        scratch_types=[
            pltpu.VMEM((M,), jnp.float32),
            pltpu.VMEM((NL1,), jnp.float32),
            pltpu.VMEM((256,), jnp.float32),
            pltpu.VMEM((K * 16,), jnp.float32),
            pltpu.VMEM((K * 16,), jnp.int32),
            pltpu.VMEM((K,), jnp.int32),
            pltpu.VMEM((K,), jnp.float32),
            pltpu.VMEM((K, 128), jnp.float32),
            pltpu.VMEM((K * DE,), jnp.float32),
            pltpu.SemaphoreType.DMA,
        ],
        compiler_params=pltpu.CompilerParams(needs_layout_passes=False),
    )
    return f(scores, em_V_flat)


def _attn_kernel(x_ref, vt_ref, ts_ref, wq_ref, bq_ref, wo_ref, bo_ref, out_ref):
    qc = lax.dot_general(x_ref[...], wq_ref[...], (((1,), (1,)), ((), ())),
                         preferred_element_type=jnp.float32,
                         precision=lax.Precision.HIGHEST) + bq_ref[...]
    vt = vt_ref[...]
    ts = ts_ref[...]
    logits = jnp.sum(qc[:, None, :] * vt, axis=-1) * (DE ** -0.5) + ts
    neg = ts == NEG_INF
    logits = jnp.where(neg, NEG_INF, logits)
    mx = jnp.max(logits, axis=-1, keepdims=True)
    mxs = jnp.where(mx == NEG_INF, 0.0, mx)
    e = jnp.where(neg, 0.0, jnp.exp(logits - mxs))
    den = jnp.sum(e, axis=-1, keepdims=True)
    attn = jnp.where(den > 0, e / den, 0.0)
    out = jnp.sum(attn[:, :, None] * vt, axis=1)
    out_ref[...] = lax.dot_general(out, wo_ref[...], (((1,), (1,)), ((), ())),
                                   preferred_element_type=jnp.float32,
                         precision=lax.Precision.HIGHEST) + bo_ref[...]


def _attn_call(x, vtop, tks, W_q_cross_w, W_q_cross_b, W_o_cross_w, W_o_cross_b):
    return pl.pallas_call(
        _attn_kernel,
        out_shape=jax.ShapeDtypeStruct((BS, D), jnp.float32),
    )(x, vtop, tks, W_q_cross_w, W_q_cross_b.reshape(1, DE),
      W_o_cross_w, W_o_cross_b.reshape(1, D))


def kernel(x, y_wm, em_K, em_V, em_S,
           W_q_em_w, W_q_em_b, W_q_cross_w, W_q_cross_b,
           W_o_cross_w, W_o_cross_b):
    scores = _scores_call(x, y_wm, W_q_em_w, W_q_em_b, em_S, em_K)
    tks, vtop = _topk_gather_call(scores, em_V.reshape(BS * M // 2, 128))
    return _attn_call(x, vtop.reshape(BS, K, DE), tks, W_q_cross_w,
                      W_q_cross_b, W_o_cross_w, W_o_cross_b)
